# baseline (speedup 1.0000x reference)
import jax
import jax.numpy as jnp
from jax import lax
from jax.experimental import pallas as pl
from jax.experimental.pallas import tpu as pltpu
from jax.experimental.pallas import tpu_sc as plsc

NUM_EMOTIONS = 1000
HIDDEN_DIM = 768
MAX_SEQ_LEN = 32
BATCH = 4096

NC = 2
NS = 16
NW = NC * NS
BPW = BATCH // NW
PR = 2
NPAIR = BPW // PR
NBUF = 2
ROUNDS = NPAIR // NBUF

EPAD = 1024
MB = 512


def _rows_body(cond_hbm, ids2_hbm, out_h_hbm, idx_v, buf_v, gsems, wsems):
    wid = lax.axis_index("s") * NC + lax.axis_index("c")
    base = wid * BPW

    pltpu.sync_copy(ids2_hbm.at[pl.ds(wid * NPAIR, NPAIR)], idx_v)

    def start_gather(p, b):
        pltpu.async_copy(cond_hbm.at[idx_v.at[p]], buf_v.at[b],
                         gsems.at[b])

    def wait_gather(p, b):
        pltpu.make_async_copy(cond_hbm.at[idx_v.at[p]],
                              buf_v.at[b], gsems.at[b]).wait()

    def start_write(p, b):
        pltpu.async_copy(buf_v.at[b],
                         out_h_hbm.at[pl.ds(base + p * PR, PR)],
                         wsems.at[b])

    def wait_write(p, b):
        pltpu.make_async_copy(buf_v.at[b],
                              out_h_hbm.at[pl.ds(base + p * PR, PR)],
                              wsems.at[b]).wait()

    start_gather(0, 0)

    def round_body(o, _):
        for b in range(NBUF):
            p = o * NBUF + b
            wait_gather(p, b)
            start_write(p, b)

            @pl.when(p >= 1)
            def _():
                wait_write(p - 1, (b - 1) % NBUF)

            @pl.when(p + 1 < NPAIR)
            def _():
                start_gather(p + 1, (b + 1) % NBUF)
        return _

    lax.fori_loop(0, ROUNDS, round_body, None)

    wait_write(NPAIR - 1, (NPAIR - 1) % NBUF)


def _masks_tc_body(ids_ref, masks_ref, out_ref):
    onehot = (ids_ref[:, :1] ==
              lax.broadcasted_iota(jnp.int32, (MB, EPAD), 1)
              ).astype(jnp.float32)
    prod = jax.lax.dot_general(onehot, masks_ref[...],
                               (((1,), (0,)), ((), ())),
                               preferred_element_type=jnp.float32)
    out_ref[...] = prod.astype(jnp.int32)


@jax.jit
def _launch(cond, masksf, ids):
    rows = pl.kernel(
        _rows_body,
        out_type=jax.ShapeDtypeStruct((BATCH, MAX_SEQ_LEN, HIDDEN_DIM),
                                      jnp.float32),
        mesh=plsc.VectorSubcoreMesh(core_axis_name="c", subcore_axis_name="s"),
        scratch_types=[
            pltpu.VMEM((NPAIR, PR), jnp.int32),
            pltpu.VMEM((NBUF, PR, MAX_SEQ_LEN, HIDDEN_DIM), jnp.float32),
            pltpu.SemaphoreType.DMA((NBUF,)),
            pltpu.SemaphoreType.DMA((NBUF,)),
        ],
    )
    masks_out = pl.pallas_call(
        _masks_tc_body,
        out_shape=jax.ShapeDtypeStruct((BATCH, MAX_SEQ_LEN), jnp.int32),
        grid=(BATCH // MB,),
        in_specs=[
            pl.BlockSpec((MB, 1), lambda i: (i, 0)),
            pl.BlockSpec((EPAD, MAX_SEQ_LEN), lambda i: (0, 0)),
        ],
        out_specs=pl.BlockSpec((MB, MAX_SEQ_LEN), lambda i: (i, 0)),
    )(jnp.reshape(ids, (BATCH, 1)), masksf)
    return rows(cond, jnp.reshape(ids, (BATCH // PR, PR))), masks_out


def kernel(conditioning, attention_masks, emotion_ids):
    masksf = jnp.pad(attention_masks.astype(jnp.float32),
                     ((0, EPAD - NUM_EMOTIONS), (0, 0)))
    return _launch(conditioning, masksf, emotion_ids)


# speedup vs baseline: 1.0030x; 1.0030x over previous
import jax
import jax.numpy as jnp
from jax import lax
from jax.experimental import pallas as pl
from jax.experimental.pallas import tpu as pltpu
from jax.experimental.pallas import tpu_sc as plsc

NUM_EMOTIONS = 1000
HIDDEN_DIM = 768
MAX_SEQ_LEN = 32
BATCH = 4096

NC = 2
NS = 16
NW = NC * NS
BPW = BATCH // NW
NBUF = 4
DEPTH = 3
ROUNDS = BPW // NBUF

EPAD = 1024
MB = 512


def _rows_body(cond_hbm, ids2_hbm, out_h_hbm, idx_v, buf_v, gsems, wsems):
    wid = lax.axis_index("s") * NC + lax.axis_index("c")
    base = wid * BPW

    pltpu.sync_copy(ids2_hbm.at[pl.ds(base, BPW)], idx_v)

    def start_gather(g, b):
        pltpu.async_copy(cond_hbm.at[idx_v.at[g]], buf_v.at[b],
                         gsems.at[b])

    def wait_gather(g, b):
        pltpu.make_async_copy(cond_hbm.at[idx_v.at[g]],
                              buf_v.at[b], gsems.at[b]).wait()

    def start_write(g, b):
        pltpu.async_copy(buf_v.at[b], out_h_hbm.at[pl.ds(base + g, 1)],
                         wsems.at[b])

    def wait_write(g, b):
        pltpu.make_async_copy(buf_v.at[b], out_h_hbm.at[pl.ds(base + g, 1)],
                              wsems.at[b]).wait()

    for b in range(DEPTH):
        start_gather(b, b)

    def round_body(o, _):
        for b in range(NBUF):
            g = o * NBUF + b
            wait_gather(g, b)
            start_write(g, b)

            @pl.when(g >= 1)
            def _():
                wait_write(g - 1, (b - 1) % NBUF)

            @pl.when(g + DEPTH < BPW)
            def _():
                start_gather(g + DEPTH, (b + DEPTH) % NBUF)
        return _

    lax.fori_loop(0, ROUNDS, round_body, None)

    wait_write(BPW - 1, (BPW - 1) % NBUF)


def _masks_tc_body(ids_ref, masks_ref, out_ref):
    onehot = (ids_ref[:, :1] ==
              lax.broadcasted_iota(jnp.int32, (MB, EPAD), 1)
              ).astype(jnp.float32)
    prod = jax.lax.dot_general(onehot, masks_ref[...],
                               (((1,), (0,)), ((), ())),
                               preferred_element_type=jnp.float32)
    out_ref[...] = prod.astype(jnp.int32)


@jax.jit
def _launch(cond, masksf, ids):
    rows = pl.kernel(
        _rows_body,
        out_type=jax.ShapeDtypeStruct((BATCH, MAX_SEQ_LEN, HIDDEN_DIM),
                                      jnp.float32),
        mesh=plsc.VectorSubcoreMesh(core_axis_name="c", subcore_axis_name="s"),
        scratch_types=[
            pltpu.VMEM((BPW, 1), jnp.int32),
            pltpu.VMEM((NBUF, 1, MAX_SEQ_LEN, HIDDEN_DIM), jnp.float32),
            pltpu.SemaphoreType.DMA((NBUF,)),
            pltpu.SemaphoreType.DMA((NBUF,)),
        ],
    )
    masks_out = pl.pallas_call(
        _masks_tc_body,
        out_shape=jax.ShapeDtypeStruct((BATCH, MAX_SEQ_LEN), jnp.int32),
        grid=(BATCH // MB,),
        in_specs=[
            pl.BlockSpec((MB, 1), lambda i: (i, 0)),
            pl.BlockSpec((EPAD, MAX_SEQ_LEN), lambda i: (0, 0)),
        ],
        out_specs=pl.BlockSpec((MB, MAX_SEQ_LEN), lambda i: (i, 0)),
    )(jnp.reshape(ids, (BATCH, 1)), masksf)
    return rows(cond, jnp.reshape(ids, (BATCH, 1))), masks_out


def kernel(conditioning, attention_masks, emotion_ids):
    masksf = jnp.pad(attention_masks.astype(jnp.float32),
                     ((0, EPAD - NUM_EMOTIONS), (0, 0)))
    return _launch(conditioning, masksf, emotion_ids)
